# Initial kernel scaffold; baseline (speedup 1.0000x reference)
import jax
import jax.numpy as jnp
from jax.experimental import pallas as pl


def kernel(query_entities, query_relations, obj_entities, ent_table, rel_table):
    raise NotImplementedError("write your pallas kernel here")


# speedup vs baseline: 2.1236x; 2.1236x over previous
import functools

import jax
import jax.numpy as jnp
from jax import lax
from jax.experimental import pallas as pl
from jax.experimental.pallas import tpu as pltpu
from jax.experimental.pallas import tpu_sc as plsc

B = 16384
D = 128
CHUNK = 128
NW = 32
BPW = B // NW
NCHUNK = BPW // CHUNK


def _gather3(qe, qr, oe, ent_table, rel_table):
    mesh = plsc.VectorSubcoreMesh(core_axis_name="c", subcore_axis_name="s")
    out_type = (
        jax.ShapeDtypeStruct((B, D), jnp.float32),
        jax.ShapeDtypeStruct((B, D), jnp.float32),
        jax.ShapeDtypeStruct((B, D), jnp.float32),
    )

    @functools.partial(
        pl.kernel,
        mesh=mesh,
        out_type=out_type,
        scratch_types=[
            pltpu.VMEM((NCHUNK, CHUNK), jnp.int32),
            pltpu.VMEM((NCHUNK, CHUNK), jnp.int32),
            pltpu.VMEM((NCHUNK, CHUNK), jnp.int32),
            pltpu.VMEM((CHUNK, D), jnp.float32),
            pltpu.SemaphoreType.DMA,
        ],
    )
    def k(qe_hbm, qr_hbm, oe_hbm, ent_hbm, rel_hbm,
          out_qe, out_qr, out_oe,
          qe_v, qr_v, oe_v, rows_v, sem):
        wid = lax.axis_index("s") * 2 + lax.axis_index("c")
        row0 = wid * NCHUNK
        pltpu.sync_copy(qe_hbm.at[pl.ds(row0, NCHUNK)], qe_v)
        pltpu.sync_copy(qr_hbm.at[pl.ds(row0, NCHUNK)], qr_v)
        pltpu.sync_copy(oe_hbm.at[pl.ds(row0, NCHUNK)], oe_v)
        for j in range(NCHUNK):
            b = (row0 + j) * CHUNK
            pltpu.async_copy(ent_hbm.at[qe_v.at[j]], rows_v, sem).wait()
            pltpu.sync_copy(rows_v, out_qe.at[pl.ds(b, CHUNK)])
            pltpu.async_copy(rel_hbm.at[qr_v.at[j]], rows_v, sem).wait()
            pltpu.sync_copy(rows_v, out_qr.at[pl.ds(b, CHUNK)])
            pltpu.async_copy(ent_hbm.at[oe_v.at[j]], rows_v, sem).wait()
            pltpu.sync_copy(rows_v, out_oe.at[pl.ds(b, CHUNK)])

    return k(qe, qr, oe, ent_table, rel_table)


def kernel(query_entities, query_relations, obj_entities, ent_table, rel_table):
    qe = query_entities.astype(jnp.int32).reshape(B // CHUNK, CHUNK)
    qr = query_relations.astype(jnp.int32).reshape(B // CHUNK, CHUNK)
    oe = obj_entities.astype(jnp.int32).reshape(B // CHUNK, CHUNK)
    out_qe, out_qr, out_oe = _gather3(qe, qr, oe, ent_table, rel_table)
    return (out_qe, out_qr, out_oe)


# speedup vs baseline: 2.4229x; 1.1410x over previous
import functools

import jax
import jax.numpy as jnp
from jax import lax
from jax.experimental import pallas as pl
from jax.experimental.pallas import tpu as pltpu
from jax.experimental.pallas import tpu_sc as plsc

B = 16384
D = 128
CHUNK = 128
NW = 32
BPW = B // NW
NCHUNK = BPW // CHUNK
T = 3 * NCHUNK
NBUF = 6


def _gather3(qe, qr, oe, ent_table, rel_table):
    mesh = plsc.VectorSubcoreMesh(core_axis_name="c", subcore_axis_name="s")
    out_type = (
        jax.ShapeDtypeStruct((B, D), jnp.float32),
        jax.ShapeDtypeStruct((B, D), jnp.float32),
        jax.ShapeDtypeStruct((B, D), jnp.float32),
    )
    scratch = (
        [pltpu.VMEM((NCHUNK, CHUNK), jnp.int32)] * 3
        + [pltpu.VMEM((CHUNK, D), jnp.float32)] * NBUF
        + [pltpu.SemaphoreType.DMA] * (1 + 2 * NBUF)
    )

    @functools.partial(pl.kernel, mesh=mesh, out_type=out_type,
                       scratch_types=scratch)
    def k(qe_hbm, qr_hbm, oe_hbm, ent_hbm, rel_hbm,
          out_qe, out_qr, out_oe, *scr):
        qe_v, qr_v, oe_v = scr[0:3]
        bufs = scr[3:3 + NBUF]
        isem = scr[3 + NBUF]
        gsem = scr[4 + NBUF:4 + 2 * NBUF]
        ssem = scr[4 + 2 * NBUF:4 + 3 * NBUF]

        wid = lax.axis_index("s") * 2 + lax.axis_index("c")
        row0 = wid * NCHUNK
        c1 = pltpu.async_copy(qe_hbm.at[pl.ds(row0, NCHUNK)], qe_v, isem)
        c2 = pltpu.async_copy(qr_hbm.at[pl.ds(row0, NCHUNK)], qr_v, isem)
        c3 = pltpu.async_copy(oe_hbm.at[pl.ds(row0, NCHUNK)], oe_v, isem)
        c1.wait(); c2.wait(); c3.wait()

        tasks = []
        for iv, tab, out in ((qe_v, ent_hbm, out_qe),
                             (qr_v, rel_hbm, out_qr),
                             (oe_v, ent_hbm, out_oe)):
            for j in range(NCHUNK):
                tasks.append((iv.at[j], tab, out, (row0 + j) * CHUNK))

        gcp = [None] * T
        scp = [None] * T
        for t in range(NBUF):
            iv_row, tab, _, _ = tasks[t]
            gcp[t] = pltpu.async_copy(tab.at[iv_row], bufs[t], gsem[t])
        for t in range(T):
            b = t % NBUF
            _, _, out, off = tasks[t]
            gcp[t].wait()
            scp[t] = pltpu.async_copy(bufs[b], out.at[pl.ds(off, CHUNK)],
                                      ssem[b])
            nt = t + NBUF
            if nt < T:
                scp[t].wait()
                iv_row, tab, _, _ = tasks[nt]
                gcp[nt] = pltpu.async_copy(tab.at[iv_row], bufs[b], gsem[b])
        for t in range(T - NBUF, T):
            scp[t].wait()

    return k(qe, qr, oe, ent_table, rel_table)


def kernel(query_entities, query_relations, obj_entities, ent_table, rel_table):
    qe = query_entities.astype(jnp.int32).reshape(B // CHUNK, CHUNK)
    qr = query_relations.astype(jnp.int32).reshape(B // CHUNK, CHUNK)
    oe = obj_entities.astype(jnp.int32).reshape(B // CHUNK, CHUNK)
    out_qe, out_qr, out_oe = _gather3(qe, qr, oe, ent_table, rel_table)
    return (out_qe, out_qr, out_oe)
